# baseline (speedup 1.0000x reference)
import jax
import jax.numpy as jnp
from jax import lax
from jax.experimental import pallas as pl
from jax.experimental.pallas import tpu as pltpu
from jax.experimental.pallas import tpu_sc as plsc

B = 4096
L = 20
D = 32
MAX_TOKENS = 10000
NC = 2
NS = 16
NW = NC * NS
BPW = B // NW
LANES = 16
KG = BPW // LANES
HD = D // LANES

_SC_PARAMS = pltpu.CompilerParams(
    use_tc_tiling_on_sc=False, needs_layout_passes=False)
_MESH = plsc.VectorSubcoreMesh(core_axis_name="c", subcore_axis_name="s")


def _tok_body(tok_ids_hbm, tok_tab_hbm, out_hbm,
              tokids_v, tokidx_v, chunks_v, acc_v, c0_v, shared_tab, sems):
    wid = lax.axis_index("s") * NC + lax.axis_index("c")
    sid = lax.axis_index("s")
    base = wid * BPW

    rows_per_sub = MAX_TOKENS // NS
    pltpu.sync_copy(
        tok_tab_hbm.at[pl.ds(sid * rows_per_sub, rows_per_sub), :],
        shared_tab.at[pl.ds(sid * rows_per_sub, rows_per_sub), :])

    pltpu.sync_copy(tok_ids_hbm.at[pl.ds(base * L, BPW * L)], tokids_v)
    pltpu.sync_copy(tok_tab_hbm.at[0], c0_v)

    lane = lax.iota(jnp.int32, LANES)
    lane_l = lane * L
    cnt = [jnp.zeros((LANES,), jnp.int32) for _ in range(KG)]
    for j in range(L):
        for k in range(KG):
            idx = lane_l + (k * LANES * L + j)
            v = plsc.load_gather(tokids_v, [idx])
            cnt[k] = cnt[k] + jnp.where(v == 0, 1, 0).astype(jnp.int32)
            tokidx_v[j, pl.ds(k * LANES, LANES)] = v
    n0f = [c.astype(jnp.float32) for c in cnt]
    scale = [1.0 / jnp.maximum(jnp.float32(L) - n, 1.0) for n in n0f]

    plsc.subcore_barrier()

    def fire(j, carry):
        pltpu.make_async_copy(
            shared_tab.at[tokidx_v.at[j]], chunks_v.at[j],
            sems.at[j]).start()
        return carry
    lax.fori_loop(0, L, fire, 0)

    zero = jnp.zeros((LANES,), jnp.float32)
    for r in range(BPW):
        for h in range(HD):
            acc_v[r, pl.ds(h * LANES, LANES)] = zero

    def accum(j, carry):
        pltpu.make_async_copy(
            shared_tab.at[tokidx_v.at[j]], chunks_v.at[j],
            sems.at[j]).wait()
        for r in range(BPW):
            for h in range(HD):
                v = chunks_v[j, r, pl.ds(h * LANES, LANES)]
                plsc.addupdate(acc_v.at[r, pl.ds(h * LANES, LANES)], v)
        return carry
    lax.fori_loop(0, L, accum, 0)

    c0 = [c0_v[pl.ds(h * LANES, LANES)] for h in range(HD)]
    for k in range(KG):
        for ln in range(LANES):
            r = k * LANES + ln
            n0 = jnp.broadcast_to(n0f[k][ln], (LANES,))
            sc = jnp.broadcast_to(scale[k][ln], (LANES,))
            for h in range(HD):
                t = acc_v[r, pl.ds(h * LANES, LANES)]
                acc_v[r, pl.ds(h * LANES, LANES)] = (t - n0 * c0[h]) * sc

    pltpu.sync_copy(acc_v, out_hbm.at[pl.ds(base, BPW), :])


def _title_body(title_ids_hbm, title_tab_hbm, text_hbm, out_hbm,
                tidx_v, trows_v, text_v, out_v, sem):
    wid = lax.axis_index("s") * NC + lax.axis_index("c")
    base = wid * BPW
    pltpu.sync_copy(title_ids_hbm.at[pl.ds(base, BPW)], tidx_v)
    title_cp = pltpu.make_async_copy(title_tab_hbm.at[tidx_v], trows_v, sem)
    title_cp.start()
    pltpu.sync_copy(text_hbm.at[pl.ds(base, BPW), :], text_v)
    title_cp.wait()
    for r in range(BPW):
        for h in range(HD):
            out_v[r, pl.ds(h * LANES, LANES)] = (
                trows_v[r, pl.ds(h * LANES, LANES)])
            out_v[r, pl.ds(D + h * LANES, LANES)] = (
                text_v[r, pl.ds(h * LANES, LANES)])
    pltpu.sync_copy(out_v, out_hbm.at[pl.ds(base, BPW), :])


@jax.jit
def _run(title_ids, tok_ids_flat, title_table, token_table):
    tok_f = pl.kernel(
        _tok_body,
        out_type=jax.ShapeDtypeStruct((B, D), jnp.float32),
        mesh=_MESH,
        compiler_params=_SC_PARAMS,
        scratch_types=[
            pltpu.VMEM((BPW * L,), jnp.int32),
            pltpu.VMEM((L, BPW), jnp.int32),
            pltpu.VMEM((L, BPW, D), jnp.float32),
            pltpu.VMEM((BPW, D), jnp.float32),
            pltpu.VMEM((D,), jnp.float32),
            pltpu.VMEM_SHARED((MAX_TOKENS, D), jnp.float32),
            pltpu.SemaphoreType.DMA((L,)),
        ],
    )
    title_f = pl.kernel(
        _title_body,
        out_type=jax.ShapeDtypeStruct((B, 2 * D), jnp.float32),
        mesh=_MESH,
        compiler_params=_SC_PARAMS,
        scratch_types=[
            pltpu.VMEM((BPW,), jnp.int32),
            pltpu.VMEM((BPW, D), jnp.float32),
            pltpu.VMEM((BPW, D), jnp.float32),
            pltpu.VMEM((BPW, 2 * D), jnp.float32),
            pltpu.SemaphoreType.DMA,
        ],
    )
    text = tok_f(tok_ids_flat, token_table)
    return title_f(title_ids, title_table, text)


def kernel(title_ids, token_ids, title_table, token_table):
    return _run(title_ids.astype(jnp.int32),
                token_ids.astype(jnp.int32).reshape(-1),
                title_table, token_table)


# speedup vs baseline: 1.0512x; 1.0512x over previous
import jax
import jax.numpy as jnp
from jax import lax
from jax.experimental import pallas as pl
from jax.experimental.pallas import tpu as pltpu
from jax.experimental.pallas import tpu_sc as plsc

B = 4096
L = 20
D = 32
MAX_TOKENS = 10000
NC = 2
NS = 16
NW = NC * NS
BPW = B // NW
LANES = 16
KG = BPW // LANES
HD = D // LANES

_SC_PARAMS = pltpu.CompilerParams(
    use_tc_tiling_on_sc=False, needs_layout_passes=False)
_MESH = plsc.VectorSubcoreMesh(core_axis_name="c", subcore_axis_name="s")


def _tok_body(tok_ids_hbm, tok_tab_hbm, out_hbm,
              tokids_v, tokidx_v, chunks_v, acc_v, c0_v, shared_tab, sems):
    wid = lax.axis_index("s") * NC + lax.axis_index("c")
    sid = lax.axis_index("s")
    base = wid * BPW

    rows_per_sub = MAX_TOKENS // NS
    pltpu.sync_copy(
        tok_tab_hbm.at[pl.ds(sid * rows_per_sub, rows_per_sub), :],
        shared_tab.at[pl.ds(sid * rows_per_sub, rows_per_sub), :])

    pltpu.sync_copy(tok_ids_hbm.at[pl.ds(base * L, BPW * L)], tokids_v)
    pltpu.sync_copy(tok_tab_hbm.at[0], c0_v)

    lane = lax.iota(jnp.int32, LANES)
    lane_l = lane * L
    cnt = [jnp.zeros((LANES,), jnp.int32) for _ in range(KG)]
    for j in range(L):
        for k in range(KG):
            idx = lane_l + (k * LANES * L + j)
            v = plsc.load_gather(tokids_v, [idx])
            cnt[k] = cnt[k] + jnp.where(v == 0, 1, 0).astype(jnp.int32)
            tokidx_v[j, pl.ds(k * LANES, LANES)] = v
    n0f = [c.astype(jnp.float32) for c in cnt]
    scale = [1.0 / jnp.maximum(jnp.float32(L) - n, 1.0) for n in n0f]

    plsc.subcore_barrier()

    def fire(j, carry):
        pltpu.make_async_copy(
            shared_tab.at[tokidx_v.at[j]], chunks_v.at[j],
            sems.at[j]).start()
        return carry
    lax.fori_loop(0, L, fire, 0)

    def drain(j, carry):
        pltpu.make_async_copy(
            shared_tab.at[tokidx_v.at[j]], chunks_v.at[j],
            sems.at[j]).wait()
        return carry
    lax.fori_loop(0, L, drain, 0)

    c0 = [c0_v[pl.ds(h * LANES, LANES)] for h in range(HD)]
    PG = BPW * HD // LANES
    for g in range(PG):
        pos = [g * LANES + i for i in range(LANES)]
        init = tuple(
            chunks_v[0, p // HD, pl.ds((p % HD) * LANES, LANES)] for p in pos)

        def body(j, accs, pos=pos):
            return tuple(
                a + chunks_v[j, p // HD, pl.ds((p % HD) * LANES, LANES)]
                for a, p in zip(accs, pos))
        accs = lax.fori_loop(1, L, body, init)

        for i, p in enumerate(pos):
            r, h = p // HD, p % HD
            k, ln = r // LANES, r % LANES
            n0 = jnp.broadcast_to(n0f[k][ln], (LANES,))
            sc = jnp.broadcast_to(scale[k][ln], (LANES,))
            acc_v[r, pl.ds(h * LANES, LANES)] = (accs[i] - n0 * c0[h]) * sc

    pltpu.sync_copy(acc_v, out_hbm.at[pl.ds(base, BPW), :])


def _title_body(title_ids_hbm, title_tab_hbm, text_hbm, out_hbm,
                tidx_v, trows_v, text_v, out_v, sem):
    wid = lax.axis_index("s") * NC + lax.axis_index("c")
    base = wid * BPW
    pltpu.sync_copy(title_ids_hbm.at[pl.ds(base, BPW)], tidx_v)
    title_cp = pltpu.make_async_copy(title_tab_hbm.at[tidx_v], trows_v, sem)
    title_cp.start()
    pltpu.sync_copy(text_hbm.at[pl.ds(base, BPW), :], text_v)
    title_cp.wait()
    for r in range(BPW):
        for h in range(HD):
            out_v[r, pl.ds(h * LANES, LANES)] = (
                trows_v[r, pl.ds(h * LANES, LANES)])
            out_v[r, pl.ds(D + h * LANES, LANES)] = (
                text_v[r, pl.ds(h * LANES, LANES)])
    pltpu.sync_copy(out_v, out_hbm.at[pl.ds(base, BPW), :])


@jax.jit
def _run(title_ids, tok_ids_flat, title_table, token_table):
    tok_f = pl.kernel(
        _tok_body,
        out_type=jax.ShapeDtypeStruct((B, D), jnp.float32),
        mesh=_MESH,
        compiler_params=_SC_PARAMS,
        scratch_types=[
            pltpu.VMEM((BPW * L,), jnp.int32),
            pltpu.VMEM((L, BPW), jnp.int32),
            pltpu.VMEM((L, BPW, D), jnp.float32),
            pltpu.VMEM((BPW, D), jnp.float32),
            pltpu.VMEM((D,), jnp.float32),
            pltpu.VMEM_SHARED((MAX_TOKENS, D), jnp.float32),
            pltpu.SemaphoreType.DMA((L,)),
        ],
    )
    title_f = pl.kernel(
        _title_body,
        out_type=jax.ShapeDtypeStruct((B, 2 * D), jnp.float32),
        mesh=_MESH,
        compiler_params=_SC_PARAMS,
        scratch_types=[
            pltpu.VMEM((BPW,), jnp.int32),
            pltpu.VMEM((BPW, D), jnp.float32),
            pltpu.VMEM((BPW, D), jnp.float32),
            pltpu.VMEM((BPW, 2 * D), jnp.float32),
            pltpu.SemaphoreType.DMA,
        ],
    )
    text = tok_f(tok_ids_flat, token_table)
    return title_f(title_ids, title_table, text)


def kernel(title_ids, token_ids, title_table, token_table):
    return _run(title_ids.astype(jnp.int32),
                token_ids.astype(jnp.int32).reshape(-1),
                title_table, token_table * jnp.float32(1.0))
